# baseline (speedup 1.0000x reference)
import jax
import jax.numpy as jnp
from jax import lax
from jax.experimental import pallas as pl
from jax.experimental.pallas import tpu as pltpu
from jax.experimental.pallas import tpu_sc as plsc

DIM = 128
NUM_MODELS = 64
TEXT_DIM = 1536
BATCH = 4096

NC = 2
NS = 16
L = 16
NW = NC * NS
B_PER_TILE = BATCH // NW


def _tc_table_body(prompt_ref, p_ref, wt_ref, wcls_ref, table_ref):
    pe = lax.dot_general(
        prompt_ref[...], wt_ref[...],
        dimension_numbers=(((1,), (1,)), ((), ())),
        preferred_element_type=jnp.float32,
    )
    w = pe * wcls_ref[...]
    p = p_ref[...]
    srow = lax.dot_general(
        w, p, dimension_numbers=(((1,), (1,)), ((), ())),
        preferred_element_type=jnp.float32,
    )
    n2row = lax.dot_general(
        jnp.ones((1, DIM), jnp.float32), p * p,
        dimension_numbers=(((1,), (1,)), ((), ())),
        preferred_element_type=jnp.float32,
    )
    table_ref[...] = srow / jnp.maximum(jnp.sqrt(n2row), 1e-12)


def _sc_gather_body(table_hbm, ids_hbm, out_hbm, table_v, ids_v, out_v,
                    sem_t, sem_id):
    cid = lax.axis_index("c")
    sid = lax.axis_index("s")
    wid = sid * NC + cid
    cp_t = pltpu.async_copy(table_hbm.at[0], table_v, sem_t)
    cp_id = pltpu.async_copy(
        ids_hbm.at[pl.ds(wid * B_PER_TILE, B_PER_TILE)], ids_v, sem_id)
    cp_t.wait()
    cp_id.wait()
    for j in range(B_PER_TILE // L):
        idx = ids_v[pl.ds(j * L, L)]
        out_v[pl.ds(j * L, L)] = plsc.load_gather(table_v, [idx])
    pltpu.sync_copy(out_v, out_hbm.at[pl.ds(wid * B_PER_TILE, B_PER_TILE)])


@jax.jit
def _run(model_id, prompt_embed, P, W_text, W_cls):
    table = pl.pallas_call(
        _tc_table_body,
        out_shape=jax.ShapeDtypeStruct((1, NUM_MODELS), jnp.float32),
    )(prompt_embed.reshape(1, TEXT_DIM), P, W_text, W_cls)
    mesh = plsc.VectorSubcoreMesh(core_axis_name="c", subcore_axis_name="s",
                                  num_cores=NC, num_subcores=NS)
    return pl.kernel(
        _sc_gather_body,
        out_type=jax.ShapeDtypeStruct((BATCH,), jnp.float32),
        mesh=mesh,
        scratch_types=[
            pltpu.VMEM((NUM_MODELS,), jnp.float32),
            pltpu.VMEM((B_PER_TILE,), jnp.int32),
            pltpu.VMEM((B_PER_TILE,), jnp.float32),
            pltpu.SemaphoreType.DMA,
            pltpu.SemaphoreType.DMA,
        ],
        compiler_params=pltpu.CompilerParams(
            needs_layout_passes=False,
            skip_device_barrier=True,
            disable_bounds_checks=True,
            disable_semaphore_checks=True,
        ),
    )(table, model_id)


def kernel(model_id, prompt_embed, P, W_text, W_cls):
    return _run(model_id.astype(jnp.int32), prompt_embed, P, W_text, W_cls)


# speedup vs baseline: 4.0953x; 4.0953x over previous
import jax
import jax.numpy as jnp
from jax import lax
from jax.experimental import pallas as pl
from jax.experimental.pallas import tpu as pltpu

DIM = 128
NUM_MODELS = 64
TEXT_DIM = 1536
BATCH = 4096
ROWS = BATCH // 128
NCHUNK = 4
CH = TEXT_DIM // NCHUNK


def _tc_body(ids_ref, prompt_ref, p_ref, wt_hbm, wcls_ref, out_ref,
             wt_buf, sems):
    def chunk_copy(k, slot):
        return pltpu.make_async_copy(
            wt_hbm.at[:, pl.ds(k * CH, CH)], wt_buf.at[slot], sems.at[slot])

    chunk_copy(0, 0).start()
    pe = jnp.zeros((1, DIM), jnp.float32)
    for k in range(NCHUNK):
        slot = k % 2
        if k + 1 < NCHUNK:
            chunk_copy(k + 1, (k + 1) % 2).start()
        chunk_copy(k, slot).wait()
        pe = pe + lax.dot_general(
            prompt_ref[:, pl.ds(k * CH, CH)], wt_buf[slot],
            dimension_numbers=(((1,), (1,)), ((), ())),
            preferred_element_type=jnp.float32,
        )
    w = pe * wcls_ref[...]
    p = p_ref[...]
    srow = lax.dot_general(
        w, p, dimension_numbers=(((1,), (1,)), ((), ())),
        preferred_element_type=jnp.float32,
    )
    n2row = lax.dot_general(
        jnp.ones((1, DIM), jnp.float32), p * p,
        dimension_numbers=(((1,), (1,)), ((), ())),
        preferred_element_type=jnp.float32,
    )
    s = srow / jnp.maximum(jnp.sqrt(n2row), 1e-12)
    sb = jnp.broadcast_to(s, (ROWS, NUM_MODELS))
    out_ref[...] = jnp.take_along_axis(sb, ids_ref[...], axis=1)


def kernel(model_id, prompt_embed, P, W_text, W_cls):
    out = pl.pallas_call(
        _tc_body,
        out_shape=jax.ShapeDtypeStruct((ROWS, 128), jnp.float32),
        in_specs=[
            pl.BlockSpec(memory_space=pltpu.VMEM),
            pl.BlockSpec(memory_space=pltpu.VMEM),
            pl.BlockSpec(memory_space=pltpu.VMEM),
            pl.BlockSpec(memory_space=pltpu.MemorySpace.HBM),
            pl.BlockSpec(memory_space=pltpu.VMEM),
        ],
        scratch_shapes=[
            pltpu.VMEM((2, DIM, CH), jnp.float32),
            pltpu.SemaphoreType.DMA((2,)),
        ],
    )(model_id.astype(jnp.int32).reshape(ROWS, 128),
      prompt_embed.reshape(1, TEXT_DIM), P, W_text, W_cls)
    return out.reshape(BATCH)
